# Initial kernel scaffold; baseline (speedup 1.0000x reference)
import jax
import jax.numpy as jnp
from jax.experimental import pallas as pl


def kernel(x, edge_attr, edge_index, params):
    raise NotImplementedError("write your pallas kernel here")


# speedup vs baseline: 2.3561x; 2.3561x over previous
import functools

import jax
import jax.numpy as jnp
from jax import lax
from jax.experimental import pallas as pl
from jax.experimental.pallas import tpu as pltpu
from jax.experimental.pallas import tpu_sc as plsc

_N = 10000
_E = 320000
_EPS = 1e-5
_L = 128
_BE = 2560
_BNB = 2000
_NW = 32
_CHUNK = 128
_NROWS = _E // _CHUNK
_BASE_T = _NROWS // _NW
_EXTRA = _NROWS - _BASE_T * _NW
_NP = 10240
_NPT = _NP // 16
_STG = 128


def _mesh():
    return plsc.VectorSubcoreMesh(core_axis_name="c", subcore_axis_name="s")


def _tile_range(wid):
    base = wid * _BASE_T + jnp.minimum(wid, _EXTRA)
    trips = _BASE_T + jnp.where(wid < _EXTRA, 1, 0)
    return base, trips


def _fill_rows(buf, nrows, value):
    v = jnp.full((16,), value, jnp.float32)

    def body(r, carry):
        for j in range(_L // 16):
            buf[r, pl.ds(j * 16, 16)] = v
        return carry

    lax.fori_loop(0, nrows, body, 0)


def _sc_gather(table, idx2d):
    @functools.partial(
        pl.kernel,
        mesh=_mesh(),
        out_type=jax.ShapeDtypeStruct((_E, _L), jnp.float32),
        scratch_types=[
            pltpu.VMEM((_CHUNK,), jnp.int32),
            pltpu.VMEM((_CHUNK, _L), jnp.float32),
            pltpu.SemaphoreType.DMA,
        ],
    )
    def k(table_hbm, idx_hbm, out_hbm, idx_v, rows_v, sem):
        cid = lax.axis_index("c")
        sid = lax.axis_index("s")
        wid = sid * 2 + cid
        base, trips = _tile_range(wid)

        def body(i, carry):
            r = base + i
            pltpu.sync_copy(idx_hbm.at[r], idx_v)
            pltpu.async_copy(table_hbm.at[idx_v], rows_v, sem).wait()
            pltpu.sync_copy(rows_v, out_hbm.at[pl.ds(r * _CHUNK, _CHUNK)])
            return carry

        lax.fori_loop(0, trips, body, 0)

    return k(table, idx2d)


def _scatter_core(idx_hbm, out_hbm, idx_v, rows_v, stage_v, acc, load_chunk):
    cid = lax.axis_index("c")
    sid = lax.axis_index("s")
    wid = sid * 2 + cid
    base, trips = _tile_range(wid)

    _fill_rows(stage_v, _STG, 0.0)
    for p in range(_NPT // _STG):
        pltpu.sync_copy(stage_v, acc.at[pl.ds(sid * _NPT + p * _STG, _STG)])
    plsc.subcore_barrier()

    def body(i, carry):
        r = base + i
        pltpu.sync_copy(idx_hbm.at[r], idx_v)
        load_chunk(r)
        pltpu.sync_copy(rows_v, acc.at[idx_v], add=True)
        return carry

    lax.fori_loop(0, trips, body, 0)
    plsc.subcore_barrier()

    for p in range(_NPT // _STG):
        r0 = sid * _NPT + p * _STG
        pltpu.sync_copy(acc.at[pl.ds(r0, _STG)], stage_v)
        pltpu.sync_copy(stage_v, out_hbm.at[cid, pl.ds(r0, _STG)])


def _sc_scatter_add(rows, idx2d):
    @functools.partial(
        pl.kernel,
        mesh=_mesh(),
        out_type=jax.ShapeDtypeStruct((2, _NP, _L), jnp.float32),
        scratch_types=[
            pltpu.VMEM((_CHUNK,), jnp.int32),
            pltpu.VMEM((_CHUNK, _L), jnp.float32),
            pltpu.VMEM((_STG, _L), jnp.float32),
            pltpu.VMEM_SHARED((_NP, _L), jnp.float32),
            pltpu.SemaphoreType.DMA,
        ],
    )
    def k(rows_hbm, idx_hbm, out_hbm, idx_v, rows_v, stage_v, acc, sem):
        def load_chunk(r):
            pltpu.sync_copy(rows_hbm.at[pl.ds(r * _CHUNK, _CHUNK)], rows_v)

        _scatter_core(idx_hbm, out_hbm, idx_v, rows_v, stage_v, acc, load_chunk)

    return k(rows, idx2d)


def _sc_count(idx2d):
    @functools.partial(
        pl.kernel,
        mesh=_mesh(),
        out_type=jax.ShapeDtypeStruct((2, _NP, _L), jnp.float32),
        scratch_types=[
            pltpu.VMEM((_CHUNK,), jnp.int32),
            pltpu.VMEM((_CHUNK, _L), jnp.float32),
            pltpu.VMEM((_STG, _L), jnp.float32),
            pltpu.VMEM_SHARED((_NP, _L), jnp.float32),
            pltpu.SemaphoreType.DMA,
        ],
    )
    def k(idx_hbm, out_hbm, idx_v, rows_v, stage_v, acc, sem):
        _fill_rows(rows_v, _CHUNK, 1.0)

        def load_chunk(r):
            pass

        _scatter_core(idx_hbm, out_hbm, idx_v, rows_v, stage_v, acc, load_chunk)

    return k(idx2d)


def _lin_relu_stats(xv, wt, b, br):
    r, kdim = xv.shape

    def body(x_ref, w_ref, b_ref, h_ref, st_ref):
        h = jnp.dot(x_ref[...], w_ref[...], preferred_element_type=jnp.float32)
        h = jnp.maximum(h + b_ref[...], 0.0)
        h_ref[...] = h
        st = jnp.concatenate(
            [jnp.sum(h, axis=0, keepdims=True),
             jnp.sum(h * h, axis=0, keepdims=True)], axis=0)

        @pl.when(pl.program_id(0) == 0)
        def _init():
            st_ref[...] = st

        @pl.when(pl.program_id(0) > 0)
        def _acc():
            st_ref[...] += st

    return pl.pallas_call(
        body,
        grid=(r // br,),
        in_specs=[pl.BlockSpec((br, kdim), lambda i: (i, 0)),
                  pl.BlockSpec((kdim, _L), lambda i: (0, 0)),
                  pl.BlockSpec((1, _L), lambda i: (0, 0))],
        out_specs=[pl.BlockSpec((br, _L), lambda i: (i, 0)),
                   pl.BlockSpec((2, _L), lambda i: (0, 0))],
        out_shape=[jax.ShapeDtypeStruct((r, _L), jnp.float32),
                   jax.ShapeDtypeStruct((2, _L), jnp.float32)],
    )(xv, wt, b)


def _post_lin(h, a, c, wt, b):
    def body(h_ref, a_ref, c_ref, w_ref, b_ref, y_ref):
        hn = a_ref[...] * h_ref[...] + c_ref[...]
        y_ref[...] = jnp.dot(
            hn, w_ref[...], preferred_element_type=jnp.float32) + b_ref[...]

    return pl.pallas_call(
        body,
        grid=(_N // _BNB,),
        in_specs=[pl.BlockSpec((_BNB, _L), lambda i: (i, 0)),
                  pl.BlockSpec((1, _L), lambda i: (0, 0)),
                  pl.BlockSpec((1, _L), lambda i: (0, 0)),
                  pl.BlockSpec((_L, _L), lambda i: (0, 0)),
                  pl.BlockSpec((1, _L), lambda i: (0, 0))],
        out_specs=pl.BlockSpec((_BNB, _L), lambda i: (i, 0)),
        out_shape=jax.ShapeDtypeStruct((_N, _L), jnp.float32),
    )(h, a, c, wt, b)


def _mp_edge(xi, xj, ep, ap, cp, w1t, w2t, w3t, b):
    def body(xi_ref, xj_ref, ep_ref, ap_ref, cp_ref, w1_ref, w2_ref, w3_ref,
             b_ref, eh_ref, st_ref):
        epn = ap_ref[...] * ep_ref[...] + cp_ref[...]
        acc = jnp.dot(xi_ref[...], w1_ref[...], preferred_element_type=jnp.float32)
        acc = acc + jnp.dot(xj_ref[...], w2_ref[...], preferred_element_type=jnp.float32)
        acc = acc + jnp.dot(epn, w3_ref[...], preferred_element_type=jnp.float32)
        h = jnp.maximum(acc + b_ref[...], 0.0)
        eh_ref[...] = h
        st = jnp.concatenate(
            [jnp.sum(h, axis=0, keepdims=True),
             jnp.sum(h * h, axis=0, keepdims=True)], axis=0)

        @pl.when(pl.program_id(0) == 0)
        def _init():
            st_ref[...] = st

        @pl.when(pl.program_id(0) > 0)
        def _acc():
            st_ref[...] += st

    vec = pl.BlockSpec((1, _L), lambda i: (0, 0))
    mat = pl.BlockSpec((_L, _L), lambda i: (0, 0))
    blk = pl.BlockSpec((_BE, _L), lambda i: (i, 0))
    return pl.pallas_call(
        body,
        grid=(_E // _BE,),
        in_specs=[blk, blk, blk, vec, vec, mat, mat, mat, vec],
        out_specs=[blk, pl.BlockSpec((2, _L), lambda i: (0, 0))],
        out_shape=[jax.ShapeDtypeStruct((_E, _L), jnp.float32),
                   jax.ShapeDtypeStruct((2, _L), jnp.float32)],
    )(xi, xj, ep, ap, cp, w1t, w2t, w3t, b)


def _mp_node(xi, eh, ae, ce, w1t, w2t, b):
    def body(xi_ref, eh_ref, ae_ref, ce_ref, w1_ref, w2_ref, b_ref,
             mh_ref, st_ref):
        en = ae_ref[...] * eh_ref[...] + ce_ref[...]
        acc = jnp.dot(xi_ref[...], w1_ref[...], preferred_element_type=jnp.float32)
        acc = acc + jnp.dot(en, w2_ref[...], preferred_element_type=jnp.float32)
        h = jnp.maximum(acc + b_ref[...], 0.0)
        mh_ref[...] = h
        st = jnp.concatenate(
            [jnp.sum(h, axis=0, keepdims=True),
             jnp.sum(h * h, axis=0, keepdims=True)], axis=0)

        @pl.when(pl.program_id(0) == 0)
        def _init():
            st_ref[...] = st

        @pl.when(pl.program_id(0) > 0)
        def _acc():
            st_ref[...] += st

    vec = pl.BlockSpec((1, _L), lambda i: (0, 0))
    mat = pl.BlockSpec((_L, _L), lambda i: (0, 0))
    blk = pl.BlockSpec((_BE, _L), lambda i: (i, 0))
    return pl.pallas_call(
        body,
        grid=(_E // _BE,),
        in_specs=[blk, blk, vec, vec, mat, mat, vec],
        out_specs=[blk, pl.BlockSpec((2, _L), lambda i: (0, 0))],
        out_shape=[jax.ShapeDtypeStruct((_E, _L), jnp.float32),
                   jax.ShapeDtypeStruct((2, _L), jnp.float32)],
    )(xi, eh, ae, ce, w1t, w2t, b)


def _node_affine(s2, cnt2, am, cm):
    def body(s_ref, c_ref, am_ref, cm_ref, o_ref):
        s = s_ref[0] + s_ref[1]
        cnt = c_ref[0] + c_ref[1]
        o_ref[...] = am_ref[...] * s + cm_ref[...] * cnt

    cube = pl.BlockSpec((2, _BNB, _L), lambda i: (0, i, 0))
    vec = pl.BlockSpec((1, _L), lambda i: (0, 0))
    return pl.pallas_call(
        body,
        grid=(_N // _BNB,),
        in_specs=[cube, cube, vec, vec],
        out_specs=pl.BlockSpec((_BNB, _L), lambda i: (i, 0)),
        out_shape=jax.ShapeDtypeStruct((_N, _L), jnp.float32),
    )(s2, cnt2, am, cm)


def _cls_out(eh, a, c, wct, bc):
    def body(eh_ref, a_ref, c_ref, w_ref, b_ref, o_ref):
        en = a_ref[...] * eh_ref[...] + c_ref[...]
        o_ref[...] = jnp.dot(
            en, w_ref[...], preferred_element_type=jnp.float32) + b_ref[...]

    return pl.pallas_call(
        body,
        grid=(_E // _BE,),
        in_specs=[pl.BlockSpec((_BE, _L), lambda i: (i, 0)),
                  pl.BlockSpec((1, _L), lambda i: (0, 0)),
                  pl.BlockSpec((1, _L), lambda i: (0, 0)),
                  pl.BlockSpec((_L, 1), lambda i: (0, 0)),
                  pl.BlockSpec((1, 1), lambda i: (0, 0))],
        out_specs=pl.BlockSpec((_BE, 1), lambda i: (i, 0)),
        out_shape=jax.ShapeDtypeStruct((_E, 1), jnp.float32),
    )(eh, a, c, wct, bc)


def _bn_affine(st, bnp, rows):
    mean = st[0] / rows
    var = st[1] / rows - mean * mean
    a = bnp["gamma"] * lax.rsqrt(var + _EPS)
    c = bnp["beta"] - mean * a
    return a.reshape(1, _L), c.reshape(1, _L)


def kernel(x, edge_attr, edge_index, params):
    ne = params["node_emb"]
    ee = params["edge_emb"]
    mpn = params["mpn"]

    src2d = edge_index[0].reshape(_NROWS, _CHUNK)
    dst2d = edge_index[1].reshape(_NROWS, _CHUNK)

    h_n, st_n = _lin_relu_stats(x, ne["lin0"]["W"].T,
                                ne["lin0"]["b"].reshape(1, _L), _BNB)
    a_n, c_n = _bn_affine(st_n, ne["bn0"], _N)
    nf = _post_lin(h_n, a_n, c_n, ne["lin1"]["W"].T,
                   ne["lin1"]["b"].reshape(1, _L))

    h_e, st_e = _lin_relu_stats(edge_attr, ee["lin0"]["W"].T,
                                ee["lin0"]["b"].reshape(1, _L), _BE)
    a_p, c_p = _bn_affine(st_e, ee["bn0"], _E)

    cnt2 = _sc_count(dst2d)

    ep = h_e
    for l in range(len(mpn)):
        pm = mpn[l]
        we = pm["e_lin"]["W"]
        wn = pm["n_lin"]["W"]
        w1t, w2t, w3t = we[:, :_L].T, we[:, _L:2 * _L].T, we[:, 2 * _L:].T
        b_e = pm["e_lin"]["b"].reshape(1, _L)
        if l == 0:
            b_e = b_e + (ee["lin1"]["b"] @ we[:, 2 * _L:].T).reshape(1, _L)
            w3t = ee["lin1"]["W"].T @ w3t

        xi = _sc_gather(nf, dst2d)
        xj = _sc_gather(nf, src2d)
        e_h, st_eh = _mp_edge(xi, xj, ep, a_p, c_p, w1t, w2t, w3t, b_e)
        a_e, c_e = _bn_affine(st_eh, pm["e_bn"], _E)
        m_h, st_m = _mp_node(xi, e_h, a_e, c_e, wn[:, :_L].T, wn[:, _L:].T,
                             pm["n_lin"]["b"].reshape(1, _L))
        a_m, c_m = _bn_affine(st_m, pm["n_bn"], _E)
        s2 = _sc_scatter_add(m_h, dst2d)
        nf = _node_affine(s2, cnt2, a_m, c_m)
        ep, a_p, c_p = e_h, a_e, c_e

    return _cls_out(ep, a_p, c_p, params["cls"]["W"].T,
                    params["cls"]["b"].reshape(1, 1))


# speedup vs baseline: 2.3638x; 1.0033x over previous
import functools

import jax
import jax.numpy as jnp
from jax import lax
from jax.experimental import pallas as pl
from jax.experimental.pallas import tpu as pltpu
from jax.experimental.pallas import tpu_sc as plsc

_N = 10000
_E = 320000
_EPS = 1e-5
_L = 128
_BE = 2560
_BNB = 2000
_NW = 32
_CHUNK = 128
_NROWS = _E // _CHUNK
_BASE_T = _NROWS // _NW
_EXTRA = _NROWS - _BASE_T * _NW
_NP = 10240
_NPT = _NP // 16
_STG = 128


def _mesh():
    return plsc.VectorSubcoreMesh(core_axis_name="c", subcore_axis_name="s")


def _tile_range(wid):
    base = wid * _BASE_T + jnp.minimum(wid, _EXTRA)
    trips = _BASE_T + jnp.where(wid < _EXTRA, 1, 0)
    return base, trips


def _fill_rows(buf, nrows, value):
    v = jnp.full((16,), value, jnp.float32)

    def body(r, carry):
        for j in range(_L // 16):
            buf[r, pl.ds(j * 16, 16)] = v
        return carry

    lax.fori_loop(0, nrows, body, 0)


def _sc_gather2(table, idx_a, idx_b):
    @functools.partial(
        pl.kernel,
        mesh=_mesh(),
        out_type=[jax.ShapeDtypeStruct((_E, _L), jnp.float32),
                  jax.ShapeDtypeStruct((_E, _L), jnp.float32)],
        scratch_types=[
            pltpu.VMEM((_CHUNK,), jnp.int32),
            pltpu.VMEM((_CHUNK, _L), jnp.float32),
            pltpu.SemaphoreType.DMA,
        ],
    )
    def k(table_hbm, ia_hbm, ib_hbm, oa_hbm, ob_hbm, idx_v, rows_v, sem):
        cid = lax.axis_index("c")
        sid = lax.axis_index("s")
        wid = sid * 2 + cid
        base, trips = _tile_range(wid)

        def body(i, carry):
            r = base + i
            pltpu.sync_copy(ia_hbm.at[r], idx_v)
            pltpu.async_copy(table_hbm.at[idx_v], rows_v, sem).wait()
            pltpu.sync_copy(rows_v, oa_hbm.at[pl.ds(r * _CHUNK, _CHUNK)])
            pltpu.sync_copy(ib_hbm.at[r], idx_v)
            pltpu.async_copy(table_hbm.at[idx_v], rows_v, sem).wait()
            pltpu.sync_copy(rows_v, ob_hbm.at[pl.ds(r * _CHUNK, _CHUNK)])
            return carry

        lax.fori_loop(0, trips, body, 0)

    return k(table, idx_a, idx_b)


def _scatter_core(idx_hbm, out_hbm, idx_v, rows_v, stage_v, acc, load_chunk):
    cid = lax.axis_index("c")
    sid = lax.axis_index("s")
    wid = sid * 2 + cid
    base, trips = _tile_range(wid)

    _fill_rows(stage_v, _STG, 0.0)
    for p in range(_NPT // _STG):
        pltpu.sync_copy(stage_v, acc.at[pl.ds(sid * _NPT + p * _STG, _STG)])
    plsc.subcore_barrier()

    def body(i, carry):
        r = base + i
        pltpu.sync_copy(idx_hbm.at[r], idx_v)
        load_chunk(r)
        pltpu.sync_copy(rows_v, acc.at[idx_v], add=True)
        return carry

    lax.fori_loop(0, trips, body, 0)
    plsc.subcore_barrier()

    for p in range(_NPT // _STG):
        r0 = sid * _NPT + p * _STG
        pltpu.sync_copy(acc.at[pl.ds(r0, _STG)], stage_v)
        pltpu.sync_copy(stage_v, out_hbm.at[cid, pl.ds(r0, _STG)])


def _sc_scatter_add(rows, idx2d):
    @functools.partial(
        pl.kernel,
        mesh=_mesh(),
        out_type=jax.ShapeDtypeStruct((2, _NP, _L), jnp.float32),
        scratch_types=[
            pltpu.VMEM((_CHUNK,), jnp.int32),
            pltpu.VMEM((_CHUNK, _L), jnp.float32),
            pltpu.VMEM((_STG, _L), jnp.float32),
            pltpu.VMEM_SHARED((_NP, _L), jnp.float32),
            pltpu.SemaphoreType.DMA,
        ],
    )
    def k(rows_hbm, idx_hbm, out_hbm, idx_v, rows_v, stage_v, acc, sem):
        def load_chunk(r):
            pltpu.sync_copy(rows_hbm.at[pl.ds(r * _CHUNK, _CHUNK)], rows_v)

        _scatter_core(idx_hbm, out_hbm, idx_v, rows_v, stage_v, acc, load_chunk)

    return k(rows, idx2d)


def _sc_count(idx2d):
    @functools.partial(
        pl.kernel,
        mesh=_mesh(),
        out_type=jax.ShapeDtypeStruct((2, _NP, _L), jnp.float32),
        scratch_types=[
            pltpu.VMEM((_CHUNK,), jnp.int32),
            pltpu.VMEM((_CHUNK, _L), jnp.float32),
            pltpu.VMEM((_STG, _L), jnp.float32),
            pltpu.VMEM_SHARED((_NP, _L), jnp.float32),
            pltpu.SemaphoreType.DMA,
        ],
    )
    def k(idx_hbm, out_hbm, idx_v, rows_v, stage_v, acc, sem):
        _fill_rows(rows_v, _CHUNK, 1.0)

        def load_chunk(r):
            pass

        _scatter_core(idx_hbm, out_hbm, idx_v, rows_v, stage_v, acc, load_chunk)

    return k(idx2d)


def _lin_relu_stats(xv, wt, b, br):
    r, kdim = xv.shape

    def body(x_ref, w_ref, b_ref, h_ref, st_ref):
        h = jnp.dot(x_ref[...], w_ref[...], preferred_element_type=jnp.float32)
        h = jnp.maximum(h + b_ref[...], 0.0)
        h_ref[...] = h
        st = jnp.concatenate(
            [jnp.sum(h, axis=0, keepdims=True),
             jnp.sum(h * h, axis=0, keepdims=True)], axis=0)

        @pl.when(pl.program_id(0) == 0)
        def _init():
            st_ref[...] = st

        @pl.when(pl.program_id(0) > 0)
        def _acc():
            st_ref[...] += st

    return pl.pallas_call(
        body,
        grid=(r // br,),
        in_specs=[pl.BlockSpec((br, kdim), lambda i: (i, 0)),
                  pl.BlockSpec((kdim, _L), lambda i: (0, 0)),
                  pl.BlockSpec((1, _L), lambda i: (0, 0))],
        out_specs=[pl.BlockSpec((br, _L), lambda i: (i, 0)),
                   pl.BlockSpec((2, _L), lambda i: (0, 0))],
        out_shape=[jax.ShapeDtypeStruct((r, _L), jnp.float32),
                   jax.ShapeDtypeStruct((2, _L), jnp.float32)],
    )(xv, wt, b)


def _post_lin(h, a, c, wt, b):
    def body(h_ref, a_ref, c_ref, w_ref, b_ref, y_ref):
        hn = a_ref[...] * h_ref[...] + c_ref[...]
        y_ref[...] = jnp.dot(
            hn, w_ref[...], preferred_element_type=jnp.float32) + b_ref[...]

    return pl.pallas_call(
        body,
        grid=(_N // _BNB,),
        in_specs=[pl.BlockSpec((_BNB, _L), lambda i: (i, 0)),
                  pl.BlockSpec((1, _L), lambda i: (0, 0)),
                  pl.BlockSpec((1, _L), lambda i: (0, 0)),
                  pl.BlockSpec((_L, _L), lambda i: (0, 0)),
                  pl.BlockSpec((1, _L), lambda i: (0, 0))],
        out_specs=pl.BlockSpec((_BNB, _L), lambda i: (i, 0)),
        out_shape=jax.ShapeDtypeStruct((_N, _L), jnp.float32),
    )(h, a, c, wt, b)


def _mp_edge(xi, xj, ep, ap, cp, w1t, w2t, w3t, b):
    def body(xi_ref, xj_ref, ep_ref, ap_ref, cp_ref, w1_ref, w2_ref, w3_ref,
             b_ref, eh_ref, st_ref):
        epn = ap_ref[...] * ep_ref[...] + cp_ref[...]
        acc = jnp.dot(xi_ref[...], w1_ref[...], preferred_element_type=jnp.float32)
        acc = acc + jnp.dot(xj_ref[...], w2_ref[...], preferred_element_type=jnp.float32)
        acc = acc + jnp.dot(epn, w3_ref[...], preferred_element_type=jnp.float32)
        h = jnp.maximum(acc + b_ref[...], 0.0)
        eh_ref[...] = h
        st = jnp.concatenate(
            [jnp.sum(h, axis=0, keepdims=True),
             jnp.sum(h * h, axis=0, keepdims=True)], axis=0)

        @pl.when(pl.program_id(0) == 0)
        def _init():
            st_ref[...] = st

        @pl.when(pl.program_id(0) > 0)
        def _acc():
            st_ref[...] += st

    vec = pl.BlockSpec((1, _L), lambda i: (0, 0))
    mat = pl.BlockSpec((_L, _L), lambda i: (0, 0))
    blk = pl.BlockSpec((_BE, _L), lambda i: (i, 0))
    return pl.pallas_call(
        body,
        grid=(_E // _BE,),
        in_specs=[blk, blk, blk, vec, vec, mat, mat, mat, vec],
        out_specs=[blk, pl.BlockSpec((2, _L), lambda i: (0, 0))],
        out_shape=[jax.ShapeDtypeStruct((_E, _L), jnp.float32),
                   jax.ShapeDtypeStruct((2, _L), jnp.float32)],
    )(xi, xj, ep, ap, cp, w1t, w2t, w3t, b)


def _mp_node(xi, eh, ae, ce, w1t, w2t, b):
    def body(xi_ref, eh_ref, ae_ref, ce_ref, w1_ref, w2_ref, b_ref,
             mh_ref, st_ref):
        en = ae_ref[...] * eh_ref[...] + ce_ref[...]
        acc = jnp.dot(xi_ref[...], w1_ref[...], preferred_element_type=jnp.float32)
        acc = acc + jnp.dot(en, w2_ref[...], preferred_element_type=jnp.float32)
        h = jnp.maximum(acc + b_ref[...], 0.0)
        mh_ref[...] = h
        st = jnp.concatenate(
            [jnp.sum(h, axis=0, keepdims=True),
             jnp.sum(h * h, axis=0, keepdims=True)], axis=0)

        @pl.when(pl.program_id(0) == 0)
        def _init():
            st_ref[...] = st

        @pl.when(pl.program_id(0) > 0)
        def _acc():
            st_ref[...] += st

    vec = pl.BlockSpec((1, _L), lambda i: (0, 0))
    mat = pl.BlockSpec((_L, _L), lambda i: (0, 0))
    blk = pl.BlockSpec((_BE, _L), lambda i: (i, 0))
    return pl.pallas_call(
        body,
        grid=(_E // _BE,),
        in_specs=[blk, blk, vec, vec, mat, mat, vec],
        out_specs=[blk, pl.BlockSpec((2, _L), lambda i: (0, 0))],
        out_shape=[jax.ShapeDtypeStruct((_E, _L), jnp.float32),
                   jax.ShapeDtypeStruct((2, _L), jnp.float32)],
    )(xi, eh, ae, ce, w1t, w2t, b)


def _node_affine(s2, cnt2, am, cm):
    def body(s_ref, c_ref, am_ref, cm_ref, o_ref):
        s = s_ref[0] + s_ref[1]
        cnt = c_ref[0] + c_ref[1]
        o_ref[...] = am_ref[...] * s + cm_ref[...] * cnt

    cube = pl.BlockSpec((2, _BNB, _L), lambda i: (0, i, 0))
    vec = pl.BlockSpec((1, _L), lambda i: (0, 0))
    return pl.pallas_call(
        body,
        grid=(_N // _BNB,),
        in_specs=[cube, cube, vec, vec],
        out_specs=pl.BlockSpec((_BNB, _L), lambda i: (i, 0)),
        out_shape=jax.ShapeDtypeStruct((_N, _L), jnp.float32),
    )(s2, cnt2, am, cm)


def _cls_out(eh, a, c, wct, bc):
    def body(eh_ref, a_ref, c_ref, w_ref, b_ref, o_ref):
        en = a_ref[...] * eh_ref[...] + c_ref[...]
        o_ref[...] = jnp.dot(
            en, w_ref[...], preferred_element_type=jnp.float32) + b_ref[...]

    return pl.pallas_call(
        body,
        grid=(_E // _BE,),
        in_specs=[pl.BlockSpec((_BE, _L), lambda i: (i, 0)),
                  pl.BlockSpec((1, _L), lambda i: (0, 0)),
                  pl.BlockSpec((1, _L), lambda i: (0, 0)),
                  pl.BlockSpec((_L, 1), lambda i: (0, 0)),
                  pl.BlockSpec((1, 1), lambda i: (0, 0))],
        out_specs=pl.BlockSpec((_BE, 1), lambda i: (i, 0)),
        out_shape=jax.ShapeDtypeStruct((_E, 1), jnp.float32),
    )(eh, a, c, wct, bc)


def _bn_affine(st, bnp, rows):
    mean = st[0] / rows
    var = st[1] / rows - mean * mean
    a = bnp["gamma"] * lax.rsqrt(var + _EPS)
    c = bnp["beta"] - mean * a
    return a.reshape(1, _L), c.reshape(1, _L)


def kernel(x, edge_attr, edge_index, params):
    ne = params["node_emb"]
    ee = params["edge_emb"]
    mpn = params["mpn"]

    src2d = edge_index[0].reshape(_NROWS, _CHUNK)
    dst2d = edge_index[1].reshape(_NROWS, _CHUNK)

    h_n, st_n = _lin_relu_stats(x, ne["lin0"]["W"].T,
                                ne["lin0"]["b"].reshape(1, _L), _BNB)
    a_n, c_n = _bn_affine(st_n, ne["bn0"], _N)
    nf = _post_lin(h_n, a_n, c_n, ne["lin1"]["W"].T,
                   ne["lin1"]["b"].reshape(1, _L))

    h_e, st_e = _lin_relu_stats(edge_attr, ee["lin0"]["W"].T,
                                ee["lin0"]["b"].reshape(1, _L), _BE)
    a_p, c_p = _bn_affine(st_e, ee["bn0"], _E)

    cnt2 = _sc_count(dst2d)

    ep = h_e
    for l in range(len(mpn)):
        pm = mpn[l]
        we = pm["e_lin"]["W"]
        wn = pm["n_lin"]["W"]
        w1t, w2t, w3t = we[:, :_L].T, we[:, _L:2 * _L].T, we[:, 2 * _L:].T
        b_e = pm["e_lin"]["b"].reshape(1, _L)
        if l == 0:
            b_e = b_e + (ee["lin1"]["b"] @ we[:, 2 * _L:].T).reshape(1, _L)
            w3t = ee["lin1"]["W"].T @ w3t

        xi, xj = _sc_gather2(nf, dst2d, src2d)
        e_h, st_eh = _mp_edge(xi, xj, ep, a_p, c_p, w1t, w2t, w3t, b_e)
        a_e, c_e = _bn_affine(st_eh, pm["e_bn"], _E)
        if l + 1 < len(mpn):
            m_h, st_m = _mp_node(xi, e_h, a_e, c_e, wn[:, :_L].T,
                                 wn[:, _L:].T, pm["n_lin"]["b"].reshape(1, _L))
            a_m, c_m = _bn_affine(st_m, pm["n_bn"], _E)
            s2 = _sc_scatter_add(m_h, dst2d)
            nf = _node_affine(s2, cnt2, a_m, c_m)
        ep, a_p, c_p = e_h, a_e, c_e

    return _cls_out(ep, a_p, c_p, params["cls"]["W"].T,
                    params["cls"]["b"].reshape(1, 1))
